# Initial kernel scaffold; baseline (speedup 1.0000x reference)
import jax
import jax.numpy as jnp
from jax.experimental import pallas as pl


def kernel(x_lnc, x_mi, ei_ll, ei_mm, ei_lm, Wl_ll, bl_ll, Wr_ll, br_ll, att_ll, bias_ll, Wl_mm, bl_mm, Wr_mm, br_mm, att_mm, bias_mm, Wl_lm, bl_lm, Wr_lm, br_lm, att_lm, bias_lm):
    raise NotImplementedError("write your pallas kernel here")


# speedup vs baseline: 18.5924x; 18.5924x over previous
import functools

import jax
import jax.numpy as jnp
from jax import lax
from jax.experimental import pallas as pl
from jax.experimental.pallas import tpu as pltpu
from jax.experimental.pallas import tpu_sc as plsc

H = 8
C = 16
D = 128
N = 10000
E = 320000
NC = 2
NS = 16
NW = NC * NS
EPW = E // NW
B = 32
NCHUNK = EPW // B
TAIL = EPW - NCHUNK * B
NPAD = 10240
DENR = 640
ACCR = NPAD + DENR
RPT = NPAD // NS
DPT = DENR // NS
ZROWS = 40


def _proj_body(x_ref, w_ref, b_ref, o_ref):
    x = x_ref[0]
    w = w_ref[0]
    o_ref[0] = lax.dot_general(
        x, w, (((1,), (1,)), ((), ())),
        preferred_element_type=jnp.float32) + b_ref[0]


def _project(x2, w_all, b_all):
    bm = 1000
    return pl.pallas_call(
        _proj_body,
        grid=(6, N // bm),
        in_specs=[
            pl.BlockSpec((1, bm, D), lambda r, i: ((r // 2) % 2 + r // 5, i, 0)),
            pl.BlockSpec((1, D, D), lambda r, i: (r, 0, 0)),
            pl.BlockSpec((1, 1, D), lambda r, i: (r, 0, 0)),
        ],
        out_specs=pl.BlockSpec((1, bm, D), lambda r, i: (r, i, 0)),
        out_shape=jax.ShapeDtypeStruct((6, N, D), jnp.float32),
    )(x2, w_all, b_all)


_MESH = plsc.VectorSubcoreMesh(core_axis_name="c", subcore_axis_name="s")


@functools.partial(
    pl.kernel,
    out_type=[
        jax.ShapeDtypeStruct((3 * 2 * NPAD, D), jnp.float32),
        jax.ShapeDtypeStruct((3 * 2 * DENR, D), jnp.float32),
    ],
    mesh=_MESH,
    scratch_types=[
        pltpu.VMEM((B,), jnp.int32),
        pltpu.VMEM((B,), jnp.int32),
        pltpu.VMEM((TAIL,), jnp.int32),
        pltpu.VMEM((TAIL,), jnp.int32),
        pltpu.VMEM((2 * B,), jnp.int32),
        pltpu.VMEM((2 * TAIL,), jnp.int32),
        pltpu.VMEM((B,), jnp.float32),
        pltpu.VMEM((B, D), jnp.float32),
        pltpu.VMEM((B, D), jnp.float32),
        pltpu.VMEM((2 * B, D), jnp.float32),
        pltpu.VMEM((8, D), jnp.float32),
        pltpu.VMEM((H, C), jnp.float32),
        pltpu.VMEM((C,), jnp.float32),
        pltpu.VMEM((ZROWS, D), jnp.float32),
        pltpu.VMEM_SHARED((ACCR, D), jnp.float32),
        pltpu.SemaphoreType.DMA,
    ],
    compiler_params=pltpu.CompilerParams(needs_layout_passes=False),
)
def _sc_pass(tl_ll, tr_ll, tl_mm, tr_mm, tl_lm, tr_lm,
             src_ll, dst_ll, src_mm, dst_mm, src_lm, dst_lm,
             att_all, out, outd,
             src_v, dst_v, src_t, dst_t, idx2_v, idx2t_v, dstf_v,
             xl_v, xr_v, msg_v, att_v, pvec, svec, zbuf, acc, gsem):
    cid = lax.axis_index("c")
    tid = lax.axis_index("s")
    wid = tid * NC + cid

    pltpu.sync_copy(att_all, att_v)

    z16 = jnp.zeros((16,), jnp.float32)
    lane = lax.iota(jnp.int32, 16)
    low8 = lane < 8
    lane7 = jnp.bitwise_and(lane, 7)
    rowsel = lane7
    khalf = jnp.bitwise_and(lane, 8)
    cols = [khalf + k for k in range(8)]
    hi_idx = rowsel + 8
    fulls = [jnp.full((16,), 8 + h, jnp.int32) for h in range(H)]

    def zb(j, carry):
        for k in range(D // 16):
            zbuf[j, pl.ds(k * 16, 16)] = z16
        return carry

    lax.fori_loop(0, ZROWS, zb, 0)

    def zero_acc_stripe():
        base = tid * (RPT + DPT)
        for m in range((RPT + DPT) // ZROWS):
            pltpu.sync_copy(zbuf, acc.at[pl.ds(base + m * ZROWS, ZROWS)])

    zero_acc_stripe()

    for r, (tl_e, tr_e, src_e, dst_e) in enumerate(
            [(tl_ll, tr_ll, src_ll, dst_ll),
             (tl_mm, tr_mm, src_mm, dst_mm),
             (tl_lm, tr_lm, src_lm, dst_lm)]):
        plsc.subcore_barrier()
        atts = [att_v[r, pl.ds(h * 16, 16)] for h in range(H)]
        base0 = wid * EPW

        def make_edge(exoff):
            def edge(i, ecarry):
                sls = []
                for h in range(H):
                    sl = xl_v[i, pl.ds(h * 16, 16)]
                    sr = xr_v[i, pl.ds(h * 16, 16)]
                    sls.append(sl)
                    s = sl + sr
                    pvec[h, :] = jnp.maximum(s, 0.2 * s) * atts[h]
                ssum = plsc.load_gather(pvec, [rowsel, cols[0]])
                for k in range(1, 8):
                    ssum = ssum + plsc.load_gather(pvec, [rowsel, cols[k]])
                svec[...] = ssum
                alpha8 = ssum + plsc.load_gather(svec, [hi_idx])
                ex8 = jnp.exp(alpha8)
                svec[...] = ex8
                fulli = jnp.full((16,), i, jnp.int32)
                dv = plsc.bitcast(plsc.load_gather(dstf_v, [fulli]),
                                  jnp.int32)
                dlow = jnp.bitwise_and(dv, 15)
                tsel = lax.shift_right_logical(dlow, 1)
                odd = jnp.bitwise_and(dlow, 1) == 1
                exlo = jnp.where(low8, ex8, z16)
                exhi = jnp.where(low8, z16, plsc.load_gather(svec, [lane7]))
                cand = jnp.where(odd, exhi, exlo)
                svec[...] = exlo + exhi
                for k in range(8):
                    msg_v[exoff + i, pl.ds(k * 16, 16)] = jnp.where(
                        tsel == k, cand, z16)
                for h in range(H):
                    evh = plsc.load_gather(svec, [fulls[h]])
                    msg_v[i, pl.ds(h * 16, 16)] = sls[h] * evh
                return ecarry
            return edge

        edge_main = make_edge(B)
        edge_tail = make_edge(TAIL)

        def chunk(j, carry):
            base = base0 + j * B
            pltpu.sync_copy(src_e.at[pl.ds(base, B)], src_v)
            pltpu.sync_copy(dst_e.at[pl.ds(base, B)], dst_v)
            cp1 = pltpu.async_copy(tl_e.at[src_v], xl_v, gsem)
            cp2 = pltpu.async_copy(tr_e.at[dst_v], xr_v, gsem)
            cp1.wait()
            cp2.wait()
            for g in range(B // 16):
                dvec = dst_v[pl.ds(g * 16, 16)]
                idx2_v[pl.ds(g * 16, 16)] = dvec
                idx2_v[pl.ds(B + g * 16, 16)] = (
                    NPAD + lax.shift_right_logical(dvec, 4))
                dstf_v[pl.ds(g * 16, 16)] = plsc.bitcast(dvec, jnp.float32)
            lax.fori_loop(0, B, edge_main, 0)
            pltpu.sync_copy(msg_v, acc.at[idx2_v], add=True)
            return carry

        lax.fori_loop(0, NCHUNK, chunk, 0)

        tbase = base0 + NCHUNK * B
        pltpu.sync_copy(src_e.at[pl.ds(tbase, TAIL)], src_t)
        pltpu.sync_copy(dst_e.at[pl.ds(tbase, TAIL)], dst_t)
        cp1 = pltpu.async_copy(tl_e.at[src_t], xl_v.at[pl.ds(0, TAIL)], gsem)
        cp2 = pltpu.async_copy(tr_e.at[dst_t], xr_v.at[pl.ds(0, TAIL)], gsem)
        cp1.wait()
        cp2.wait()
        dvec = dst_t[...]
        idx2t_v[pl.ds(0, 16)] = dvec
        idx2t_v[pl.ds(16, 16)] = NPAD + lax.shift_right_logical(dvec, 4)
        dstf_v[pl.ds(0, 16)] = plsc.bitcast(dvec, jnp.float32)
        lax.fori_loop(0, TAIL, edge_tail, 0)
        pltpu.sync_copy(msg_v.at[pl.ds(0, 2 * TAIL)], acc.at[idx2t_v],
                        add=True)

        plsc.subcore_barrier()
        obase = (r * 2 + cid) * NPAD
        for m in range(RPT // ZROWS):
            row = tid * RPT + m * ZROWS
            pltpu.sync_copy(acc.at[pl.ds(row, ZROWS)],
                            out.at[pl.ds(obase + row, ZROWS)])
            pltpu.sync_copy(zbuf, acc.at[pl.ds(row, ZROWS)])
        drow0 = NPAD + tid * DPT
        pltpu.sync_copy(acc.at[pl.ds(drow0, DPT)],
                        outd.at[pl.ds((r * 2 + cid) * DENR + tid * DPT, DPT)])
        pltpu.sync_copy(zbuf, acc.at[pl.ds(drow0, DPT)])


def _fin_body(acc_ref, den_ref, bll_ref, bmm_ref, blm_ref, o1_ref, o2_ref):
    s = acc_ref[:, 0] + acc_ref[:, 1]
    d = den_ref[:, 0] + den_ref[:, 1]
    r8 = lax.broadcasted_iota(jnp.int32, (H, D), 0)
    c128 = lax.broadcasted_iota(jnp.int32, (H, D), 1)
    expand = jnp.where(c128 // C == r8, jnp.float32(1), jnp.float32(0))
    outs = []
    for r in range(3):
        drep = lax.dot_general(d[r], expand, (((1,), (0,)), ((), ())),
                               preferred_element_type=jnp.float32)
        outs.append(s[r] / (drep + 1e-16))
    o1_ref[...] = outs[0] + bll_ref[...]
    o2_ref[...] = 0.5 * (outs[1] + bmm_ref[...] + outs[2] + blm_ref[...])


def _finalize(acc, den4, b_ll, b_mm, b_lm):
    bm = 1000
    return pl.pallas_call(
        _fin_body,
        grid=(N // bm,),
        in_specs=[
            pl.BlockSpec((3, 2, bm, D), lambda i: (0, 0, i, 0)),
            pl.BlockSpec((3, 2, bm, H), lambda i: (0, 0, i, 0)),
            pl.BlockSpec((1, D), lambda i: (0, 0)),
            pl.BlockSpec((1, D), lambda i: (0, 0)),
            pl.BlockSpec((1, D), lambda i: (0, 0)),
        ],
        out_specs=[
            pl.BlockSpec((bm, D), lambda i: (i, 0)),
            pl.BlockSpec((bm, D), lambda i: (i, 0)),
        ],
        out_shape=[
            jax.ShapeDtypeStruct((N, D), jnp.float32),
            jax.ShapeDtypeStruct((N, D), jnp.float32),
        ],
    )(acc, den4, b_ll, b_mm, b_lm)


@jax.jit
def kernel(x_lnc, x_mi, ei_ll, ei_mm, ei_lm,
           Wl_ll, bl_ll, Wr_ll, br_ll, att_ll, bias_ll,
           Wl_mm, bl_mm, Wr_mm, br_mm, att_mm, bias_mm,
           Wl_lm, bl_lm, Wr_lm, br_lm, att_lm, bias_lm):
    x2 = jnp.stack([x_lnc, x_mi])
    w_all = jnp.stack([Wl_ll, Wr_ll, Wl_mm, Wr_mm, Wl_lm, Wr_lm])
    b_all = jnp.stack([bl_ll, br_ll, bl_mm, br_mm, bl_lm, br_lm])
    b_all = b_all.reshape(6, 1, D)
    tables = _project(x2, w_all, b_all)
    att_flat = jnp.stack([att_ll[0].reshape(D), att_mm[0].reshape(D),
                          att_lm[0].reshape(D)])
    att_all = jnp.concatenate(
        [att_flat, jnp.zeros((5, D), jnp.float32)], axis=0)
    acc, den = _sc_pass(tables[0], tables[1], tables[2], tables[3],
                        tables[4], tables[5],
                        ei_ll[0], ei_ll[1], ei_mm[0], ei_mm[1],
                        ei_lm[0], ei_lm[1], att_all)
    acc4 = acc.reshape(3, 2, NPAD, D)
    den4 = den.reshape(3, 2, NPAD, H)
    return _finalize(acc4, den4, bias_ll.reshape(1, D), bias_mm.reshape(1, D),
                     bias_lm.reshape(1, D))


# speedup vs baseline: 20.7055x; 1.1137x over previous
import functools

import jax
import jax.numpy as jnp
from jax import lax
from jax.experimental import pallas as pl
from jax.experimental.pallas import tpu as pltpu
from jax.experimental.pallas import tpu_sc as plsc

H = 8
C = 16
D = 128
N = 10000
E = 320000
NC = 2
NS = 16
NW = NC * NS
EPW = E // NW
B = 64
NCHUNK = EPW // B
TAIL = EPW - NCHUNK * B
NPAD = 10240
DENR = 640
ACCR = NPAD + DENR
RPT = NPAD // NS
DPT = DENR // NS
ZROWS = 40


def _proj_body(x_ref, w_ref, b_ref, o_ref):
    x = x_ref[0]
    w = w_ref[0]
    o_ref[0] = lax.dot_general(
        x, w, (((1,), (1,)), ((), ())),
        preferred_element_type=jnp.float32) + b_ref[0]


def _project(x2, w_all, b_all):
    bm = 1000
    return pl.pallas_call(
        _proj_body,
        grid=(6, N // bm),
        in_specs=[
            pl.BlockSpec((1, bm, D), lambda r, i: ((r // 2) % 2 + r // 5, i, 0)),
            pl.BlockSpec((1, D, D), lambda r, i: (r, 0, 0)),
            pl.BlockSpec((1, 1, D), lambda r, i: (r, 0, 0)),
        ],
        out_specs=pl.BlockSpec((1, bm, D), lambda r, i: (r, i, 0)),
        out_shape=jax.ShapeDtypeStruct((6, N, D), jnp.float32),
    )(x2, w_all, b_all)


_MESH = plsc.VectorSubcoreMesh(core_axis_name="c", subcore_axis_name="s")


@functools.partial(
    pl.kernel,
    out_type=[
        jax.ShapeDtypeStruct((3 * 2 * NPAD, D), jnp.float32),
        jax.ShapeDtypeStruct((3 * 2 * DENR, D), jnp.float32),
    ],
    mesh=_MESH,
    scratch_types=[
        pltpu.VMEM((B,), jnp.int32),
        pltpu.VMEM((B,), jnp.int32),
        pltpu.VMEM((TAIL,), jnp.int32),
        pltpu.VMEM((TAIL,), jnp.int32),
        pltpu.VMEM((2 * B,), jnp.int32),
        pltpu.VMEM((2 * TAIL,), jnp.int32),
        pltpu.VMEM((B,), jnp.float32),
        pltpu.VMEM((B, D), jnp.float32),
        pltpu.VMEM((B, D), jnp.float32),
        pltpu.VMEM((2 * B, D), jnp.float32),
        pltpu.VMEM((8, D), jnp.float32),
        pltpu.VMEM((H, C), jnp.float32),
        pltpu.VMEM((C,), jnp.float32),
        pltpu.VMEM((ZROWS, D), jnp.float32),
        pltpu.VMEM_SHARED((ACCR, D), jnp.float32),
        pltpu.SemaphoreType.DMA,
    ],
    compiler_params=pltpu.CompilerParams(needs_layout_passes=False),
)
def _sc_pass(tl_ll, tr_ll, tl_mm, tr_mm, tl_lm, tr_lm,
             src_ll, dst_ll, src_mm, dst_mm, src_lm, dst_lm,
             att_all, out, outd,
             src_v, dst_v, src_t, dst_t, idx2_v, idx2t_v, dstf_v,
             xl_v, xr_v, msg_v, att_v, pvec, svec, zbuf, acc, gsem):
    cid = lax.axis_index("c")
    tid = lax.axis_index("s")
    wid = tid * NC + cid

    pltpu.sync_copy(att_all, att_v)

    z16 = jnp.zeros((16,), jnp.float32)
    lane = lax.iota(jnp.int32, 16)
    low8 = lane < 8
    lane7 = jnp.bitwise_and(lane, 7)
    rowsel = lane7
    khalf = jnp.bitwise_and(lane, 8)
    cols = [khalf + k for k in range(8)]
    hi_idx = rowsel + 8
    fulls = [jnp.full((16,), 8 + h, jnp.int32) for h in range(H)]

    def zb(j, carry):
        for k in range(D // 16):
            zbuf[j, pl.ds(k * 16, 16)] = z16
        return carry

    lax.fori_loop(0, ZROWS, zb, 0)

    def zero_acc_stripe():
        base = tid * (RPT + DPT)
        for m in range((RPT + DPT) // ZROWS):
            pltpu.sync_copy(zbuf, acc.at[pl.ds(base + m * ZROWS, ZROWS)])

    zero_acc_stripe()

    for r, (tl_e, tr_e, src_e, dst_e) in enumerate(
            [(tl_ll, tr_ll, src_ll, dst_ll),
             (tl_mm, tr_mm, src_mm, dst_mm),
             (tl_lm, tr_lm, src_lm, dst_lm)]):
        plsc.subcore_barrier()
        atts = [att_v[r, pl.ds(h * 16, 16)] for h in range(H)]
        base0 = wid * EPW

        def make_edge(exoff):
            def edge(i, ecarry):
                sls = []
                for h in range(H):
                    sl = xl_v[i, pl.ds(h * 16, 16)]
                    sr = xr_v[i, pl.ds(h * 16, 16)]
                    sls.append(sl)
                    s = sl + sr
                    pvec[h, :] = jnp.maximum(s, 0.2 * s) * atts[h]
                ssum = plsc.load_gather(pvec, [rowsel, cols[0]])
                for k in range(1, 8):
                    ssum = ssum + plsc.load_gather(pvec, [rowsel, cols[k]])
                svec[...] = ssum
                alpha8 = ssum + plsc.load_gather(svec, [hi_idx])
                ex8 = jnp.exp(alpha8)
                svec[...] = ex8
                fulli = jnp.full((16,), i, jnp.int32)
                dv = plsc.bitcast(plsc.load_gather(dstf_v, [fulli]),
                                  jnp.int32)
                dlow = jnp.bitwise_and(dv, 15)
                tsel = lax.shift_right_logical(dlow, 1)
                odd = jnp.bitwise_and(dlow, 1) == 1
                exlo = jnp.where(low8, ex8, z16)
                exhi = jnp.where(low8, z16, plsc.load_gather(svec, [lane7]))
                cand = jnp.where(odd, exhi, exlo)
                svec[...] = exlo + exhi
                for k in range(8):
                    msg_v[exoff + i, pl.ds(k * 16, 16)] = jnp.where(
                        tsel == k, cand, z16)
                for h in range(H):
                    evh = plsc.load_gather(svec, [fulls[h]])
                    msg_v[i, pl.ds(h * 16, 16)] = sls[h] * evh
                return ecarry
            return edge

        edge_main = make_edge(B)
        edge_tail = make_edge(TAIL)

        def chunk(j, carry):
            base = base0 + j * B
            pltpu.sync_copy(src_e.at[pl.ds(base, B)], src_v)
            pltpu.sync_copy(dst_e.at[pl.ds(base, B)], dst_v)
            cp1 = pltpu.async_copy(tl_e.at[src_v], xl_v, gsem)
            cp2 = pltpu.async_copy(tr_e.at[dst_v], xr_v, gsem)
            cp1.wait()
            cp2.wait()
            for g in range(B // 16):
                dvec = dst_v[pl.ds(g * 16, 16)]
                idx2_v[pl.ds(g * 16, 16)] = dvec
                idx2_v[pl.ds(B + g * 16, 16)] = (
                    NPAD + lax.shift_right_logical(dvec, 4))
                dstf_v[pl.ds(g * 16, 16)] = plsc.bitcast(dvec, jnp.float32)
            lax.fori_loop(0, B, edge_main, 0)
            pltpu.sync_copy(msg_v, acc.at[idx2_v], add=True)
            return carry

        lax.fori_loop(0, NCHUNK, chunk, 0)

        tbase = base0 + NCHUNK * B
        pltpu.sync_copy(src_e.at[pl.ds(tbase, TAIL)], src_t)
        pltpu.sync_copy(dst_e.at[pl.ds(tbase, TAIL)], dst_t)
        cp1 = pltpu.async_copy(tl_e.at[src_t], xl_v.at[pl.ds(0, TAIL)], gsem)
        cp2 = pltpu.async_copy(tr_e.at[dst_t], xr_v.at[pl.ds(0, TAIL)], gsem)
        cp1.wait()
        cp2.wait()
        dvec = dst_t[...]
        idx2t_v[pl.ds(0, 16)] = dvec
        idx2t_v[pl.ds(16, 16)] = NPAD + lax.shift_right_logical(dvec, 4)
        dstf_v[pl.ds(0, 16)] = plsc.bitcast(dvec, jnp.float32)
        lax.fori_loop(0, TAIL, edge_tail, 0)
        pltpu.sync_copy(msg_v.at[pl.ds(0, 2 * TAIL)], acc.at[idx2t_v],
                        add=True)

        plsc.subcore_barrier()
        obase = (r * 2 + cid) * NPAD
        for m in range(RPT // ZROWS):
            row = tid * RPT + m * ZROWS
            pltpu.sync_copy(acc.at[pl.ds(row, ZROWS)],
                            out.at[pl.ds(obase + row, ZROWS)])
            pltpu.sync_copy(zbuf, acc.at[pl.ds(row, ZROWS)])
        drow0 = NPAD + tid * DPT
        pltpu.sync_copy(acc.at[pl.ds(drow0, DPT)],
                        outd.at[pl.ds((r * 2 + cid) * DENR + tid * DPT, DPT)])
        pltpu.sync_copy(zbuf, acc.at[pl.ds(drow0, DPT)])


def _fin_body(acc_ref, den_ref, bll_ref, bmm_ref, blm_ref, o1_ref, o2_ref):
    s = acc_ref[:, 0] + acc_ref[:, 1]
    d = den_ref[:, 0] + den_ref[:, 1]
    r8 = lax.broadcasted_iota(jnp.int32, (H, D), 0)
    c128 = lax.broadcasted_iota(jnp.int32, (H, D), 1)
    expand = jnp.where(c128 // C == r8, jnp.float32(1), jnp.float32(0))
    outs = []
    for r in range(3):
        drep = lax.dot_general(d[r], expand, (((1,), (0,)), ((), ())),
                               preferred_element_type=jnp.float32)
        outs.append(s[r] / (drep + 1e-16))
    o1_ref[...] = outs[0] + bll_ref[...]
    o2_ref[...] = 0.5 * (outs[1] + bmm_ref[...] + outs[2] + blm_ref[...])


def _finalize(acc, den4, b_ll, b_mm, b_lm):
    bm = 1000
    return pl.pallas_call(
        _fin_body,
        grid=(N // bm,),
        in_specs=[
            pl.BlockSpec((3, 2, bm, D), lambda i: (0, 0, i, 0)),
            pl.BlockSpec((3, 2, bm, H), lambda i: (0, 0, i, 0)),
            pl.BlockSpec((1, D), lambda i: (0, 0)),
            pl.BlockSpec((1, D), lambda i: (0, 0)),
            pl.BlockSpec((1, D), lambda i: (0, 0)),
        ],
        out_specs=[
            pl.BlockSpec((bm, D), lambda i: (i, 0)),
            pl.BlockSpec((bm, D), lambda i: (i, 0)),
        ],
        out_shape=[
            jax.ShapeDtypeStruct((N, D), jnp.float32),
            jax.ShapeDtypeStruct((N, D), jnp.float32),
        ],
    )(acc, den4, b_ll, b_mm, b_lm)


@jax.jit
def kernel(x_lnc, x_mi, ei_ll, ei_mm, ei_lm,
           Wl_ll, bl_ll, Wr_ll, br_ll, att_ll, bias_ll,
           Wl_mm, bl_mm, Wr_mm, br_mm, att_mm, bias_mm,
           Wl_lm, bl_lm, Wr_lm, br_lm, att_lm, bias_lm):
    x2 = jnp.stack([x_lnc, x_mi])
    w_all = jnp.stack([Wl_ll, Wr_ll, Wl_mm, Wr_mm, Wl_lm, Wr_lm])
    b_all = jnp.stack([bl_ll, br_ll, bl_mm, br_mm, bl_lm, br_lm])
    b_all = b_all.reshape(6, 1, D)
    tables = _project(x2, w_all, b_all)
    att_flat = jnp.stack([att_ll[0].reshape(D), att_mm[0].reshape(D),
                          att_lm[0].reshape(D)])
    att_all = jnp.concatenate(
        [att_flat, jnp.zeros((5, D), jnp.float32)], axis=0)
    acc, den = _sc_pass(tables[0], tables[1], tables[2], tables[3],
                        tables[4], tables[5],
                        ei_ll[0], ei_ll[1], ei_mm[0], ei_mm[1],
                        ei_lm[0], ei_lm[1], att_all)
    acc4 = acc.reshape(3, 2, NPAD, D)
    den4 = den.reshape(3, 2, NPAD, H)
    return _finalize(acc4, den4, bias_ll.reshape(1, D), bias_mm.reshape(1, D),
                     bias_lm.reshape(1, D))
